# baseline (speedup 1.0000x reference)
import functools

import jax
import jax.numpy as jnp
from jax import lax
from jax.experimental import pallas as pl
from jax.experimental.pallas import tpu as pltpu
from jax.experimental.pallas import tpu_sc as plsc

_DIM = 128
_HALF = _DIM // 2
_BATCH = 16384
_NC = 2
_NS = 16
_NW = _NC * _NS
_ROWS_PER_W = _BATCH // _NW
_PASS_ROWS = 256
_IDX_CH = 128
_LANES = 16

_mesh = plsc.VectorSubcoreMesh(core_axis_name="c", subcore_axis_name="s")


@functools.partial(
    pl.kernel,
    out_type=jax.ShapeDtypeStruct((_BATCH, _DIM), jnp.float32),
    mesh=_mesh,
    scratch_types=[
        pltpu.VMEM((_ROWS_PER_W // _IDX_CH, _IDX_CH), jnp.int32),
        pltpu.VMEM((_PASS_ROWS, _DIM), jnp.float32),
        pltpu.VMEM((_PASS_ROWS, _HALF), jnp.float32),
        pltpu.VMEM((_PASS_ROWS, _HALF), jnp.float32),
        pltpu.SemaphoreType.DMA,
    ],
    compiler_params=pltpu.CompilerParams(use_tc_tiling_on_sc=False),
)
def _sc_complex_diag(emb_hbm, idx_hbm, real_hbm, imag_hbm, out_hbm,
                     idx_v, emb_v, rb_v, ib_v, sem):
    wid = lax.axis_index("s") * _NC + lax.axis_index("c")
    base = wid * _ROWS_PER_W

    for j in range(_ROWS_PER_W // _IDX_CH):
        pltpu.sync_copy(idx_hbm.at[pl.ds(base + j * _IDX_CH, _IDX_CH)],
                        idx_v.at[j])

    for p in range(_ROWS_PER_W // _PASS_ROWS):
        row0 = base + p * _PASS_ROWS
        copies = [
            pltpu.async_copy(emb_hbm.at[pl.ds(row0, _PASS_ROWS)], emb_v, sem)
        ]
        for j in range(_PASS_ROWS // _IDX_CH):
            c = p * (_PASS_ROWS // _IDX_CH) + j
            dst = pl.ds(j * _IDX_CH, _IDX_CH)
            copies.append(
                pltpu.async_copy(real_hbm.at[idx_v.at[c]], rb_v.at[dst], sem))
            copies.append(
                pltpu.async_copy(imag_hbm.at[idx_v.at[c]], ib_v.at[dst], sem))
        for cp in copies:
            cp.wait()

        @pl.loop(0, _PASS_ROWS)
        def _(r):
            for ch in range(_HALF // _LANES):
                lo = pl.ds(ch * _LANES, _LANES)
                hi = pl.ds(_HALF + ch * _LANES, _LANES)
                ra = emb_v[r, lo]
                ia = emb_v[r, hi]
                rb = rb_v[r, lo]
                ib = ib_v[r, lo]
                emb_v[r, lo] = ra * rb - ia * ib
                emb_v[r, hi] = ra * ib + ia * rb

        pltpu.sync_copy(emb_v, out_hbm.at[pl.ds(row0, _PASS_ROWS)])


def kernel(embeddings, operator_idxs, real, imag):
    idx = operator_idxs.astype(jnp.int32)
    return _sc_complex_diag(embeddings, idx, real, imag)


# speedup vs baseline: 1.8705x; 1.8705x over previous
import functools

import jax
import jax.numpy as jnp
from jax import lax
from jax.experimental import pallas as pl
from jax.experimental.pallas import tpu as pltpu
from jax.experimental.pallas import tpu_sc as plsc

_DIM = 128
_HALF = _DIM // 2
_BATCH = 16384
_NUM_OPS_GRP = 12500
_NC = 2
_NS = 16
_NW = _NC * _NS
_ROWS_PER_W = _BATCH // _NW
_PASS = 256
_NPASS = _ROWS_PER_W // _PASS
_LANES = 16

_mesh = plsc.VectorSubcoreMesh(core_axis_name="c", subcore_axis_name="s")


@functools.partial(
    pl.kernel,
    out_type=jax.ShapeDtypeStruct((_BATCH, _DIM), jnp.float32),
    mesh=_mesh,
    scratch_types=[
        pltpu.VMEM((_ROWS_PER_W + _LANES,), jnp.int32),
        pltpu.VMEM((_PASS, _DIM), jnp.float32),
        pltpu.VMEM((_PASS // 8, 8, _HALF), jnp.float32),
        pltpu.VMEM((_PASS // 8, 8, _HALF), jnp.float32),
        pltpu.SemaphoreType.DMA,
    ],
)
def _sc_complex_diag(emb_hbm, idx_hbm, real_hbm, imag_hbm, out_hbm,
                     idx_v, emb_v, rb_v, ib_v, sem):
    wid = lax.axis_index("s") * _NC + lax.axis_index("c")
    base = wid * _ROWS_PER_W

    pltpu.sync_copy(idx_hbm.at[pl.ds(base, _ROWS_PER_W)],
                    idx_v.at[pl.ds(0, _ROWS_PER_W)])

    for p in range(_NPASS):
        row0 = base + p * _PASS
        emb_cp = pltpu.async_copy(emb_hbm.at[pl.ds(row0, _PASS)], emb_v, sem)

        @pl.loop(0, _PASS)
        def _(r):
            k = idx_v[pl.ds(p * _PASS + r, _LANES)][0]
            g = k >> 3
            s = k & 7
            rg = r >> 3
            rs = r & 7
            pltpu.async_copy(real_hbm.at[g, s], rb_v.at[rg, rs], sem)
            pltpu.async_copy(imag_hbm.at[g, s], ib_v.at[rg, rs], sem)

        dummy = real_hbm.at[pl.ds(0, _PASS // 8)]
        pltpu.make_async_copy(dummy, rb_v, sem).wait()
        pltpu.make_async_copy(dummy, ib_v, sem).wait()
        emb_cp.wait()

        @pl.loop(0, _PASS)
        def _(r):
            for c in range(_HALF // _LANES):
                lo = pl.ds(c * _LANES, _LANES)
                hi = pl.ds(_HALF + c * _LANES, _LANES)
                ra = emb_v[r, lo]
                ia = emb_v[r, hi]
                rb = rb_v[r >> 3, r & 7, lo]
                ib = ib_v[r >> 3, r & 7, lo]
                emb_v[r, lo] = ra * rb - ia * ib
                emb_v[r, hi] = ra * ib + ia * rb

        pltpu.sync_copy(emb_v, out_hbm.at[pl.ds(row0, _PASS)])


def kernel(embeddings, operator_idxs, real, imag):
    idx = operator_idxs.astype(jnp.int32)
    real3 = real.reshape(_NUM_OPS_GRP, 8, _HALF)
    imag3 = imag.reshape(_NUM_OPS_GRP, 8, _HALF)
    return _sc_complex_diag(embeddings, idx, real3, imag3)
